# baseline (speedup 1.0000x reference)
import jax
import jax.numpy as jnp
from jax import lax
from jax.experimental import pallas as pl
from jax.experimental.pallas import tpu as pltpu
from jax.experimental.pallas import tpu_sc as plsc

B = 16384
F = 26
VOCAB = 100000
D = 32

NC = 2
NS = 16
NW = NC * NS
N = B * F
R = N // NW
C = 1024
G = 128
KG = C // G
NCHUNK = R // C
L = 16


def _body(idx_hbm, tab_hbm, out_hbm, idx_v, gidx_v, rows_v, sem):
    wid = lax.axis_index("s") * NC + lax.axis_index("c")
    lane = lax.iota(jnp.int32, L)

    def chunk(c, carry):
        base = wid * R + c * C
        pltpu.sync_copy(idx_hbm.at[pl.ds(base, C)], idx_v)

        for k in range(KG):
            def grp(j, carry2):
                raw = idx_v[pl.ds(k * G + j * L, L)]
                pos = base + k * G + j * L + lane
                f = lax.rem(pos, F)
                gid = f * VOCAB + jnp.clip(raw, 0, VOCAB - 1)
                gidx_v[k, pl.ds(j * L, L)] = gid
                return carry2
            lax.fori_loop(0, G // L, grp, 0)

        copies = [
            pltpu.make_async_copy(
                tab_hbm.at[gidx_v.at[k]],
                rows_v.at[pl.ds(k * G, G)],
                sem,
            )
            for k in range(KG)
        ]
        for cp in copies:
            cp.start()
        for cp in copies:
            cp.wait()

        pltpu.sync_copy(rows_v, out_hbm.at[pl.ds(base, C)])
        return carry

    lax.fori_loop(0, NCHUNK, chunk, 0)


def kernel(indices, tables):
    idx_flat = indices.reshape(N).astype(jnp.int32)
    tab_flat = tables.reshape(F * VOCAB, D)
    mesh = plsc.VectorSubcoreMesh(
        core_axis_name="c", subcore_axis_name="s",
        num_cores=NC, num_subcores=NS,
    )
    f = pl.kernel(
        _body,
        out_type=jax.ShapeDtypeStruct((N, D), jnp.float32),
        mesh=mesh,
        scratch_types=[
            pltpu.VMEM((C,), jnp.int32),
            pltpu.VMEM((KG, G), jnp.int32),
            pltpu.VMEM((C, D), jnp.float32),
            pltpu.SemaphoreType.DMA,
        ],
        compiler_params=pltpu.CompilerParams(use_tc_tiling_on_sc=False),
    )
    out = f(idx_flat, tab_flat)
    return out.reshape(B, F, D)
